# baseline (speedup 1.0000x reference)
import functools

import jax
import jax.numpy as jnp
from jax import lax
from jax.experimental import pallas as pl
from jax.experimental.pallas import tpu as pltpu
from jax.experimental.pallas import tpu_sc as plsc

VOCAB = 1000000
EMBED_DIM = 64
MAX_SEQ = 2048
BATCH = 4
SEQ = 2048

NUM_CORES = 2
NUM_SUBCORES = 16
NUM_WORKERS = NUM_CORES * NUM_SUBCORES
ROWS = BATCH * SEQ
ROWS_PER_W = ROWS // NUM_WORKERS
T_PER_W = SEQ // (NUM_WORKERS // BATCH)
IDX_CHUNK = 128
LANES = 16
COL_CHUNKS = EMBED_DIM // LANES


def _make_sc_kernel():
    mesh = plsc.VectorSubcoreMesh(core_axis_name="c", subcore_axis_name="s")

    @functools.partial(
        pl.kernel,
        mesh=mesh,
        compiler_params=pltpu.CompilerParams(use_tc_tiling_on_sc=False),
        out_type=jax.ShapeDtypeStruct((ROWS, EMBED_DIM), jnp.float32),
        scratch_types=[
            pltpu.VMEM((ROWS_PER_W,), jnp.int32),
            pltpu.VMEM((ROWS_PER_W, EMBED_DIM), jnp.float32),
            pltpu.VMEM((ROWS_PER_W, EMBED_DIM), jnp.float32),
            pltpu.SemaphoreType.DMA,
        ],
    )
    def emb_kernel(x0_hbm, table_hbm, pos_hbm, out_hbm, idx_v, rows_v, pos_v, sem):
        wid = lax.axis_index("s") * NUM_CORES + lax.axis_index("c")
        base = wid * ROWS_PER_W
        t0 = (wid % (NUM_WORKERS // BATCH)) * T_PER_W

        pltpu.sync_copy(x0_hbm.at[pl.ds(base, ROWS_PER_W)], idx_v)
        pltpu.sync_copy(pos_hbm.at[pl.ds(t0, T_PER_W)], pos_v)

        cps = []
        for j in range(ROWS_PER_W // IDX_CHUNK):
            cps.append(
                pltpu.async_copy(
                    table_hbm.at[idx_v.at[pl.ds(j * IDX_CHUNK, IDX_CHUNK)]],
                    rows_v.at[pl.ds(j * IDX_CHUNK, IDX_CHUNK)],
                    sem,
                )
            )

        @pl.when(t0 == 0)
        def _zero_pos_row0():
            for c in range(COL_CHUNKS):
                pos_v[0, pl.ds(c * LANES, LANES)] = jnp.zeros(
                    (LANES,), jnp.float32
                )

        for cp in cps:
            cp.wait()

        def group_body(g, _):
            idxg = idx_v[pl.ds(g * LANES, LANES)]
            maskv = jnp.where(idxg == 0, 0.0, 1.0).astype(jnp.float32)
            for rr in range(LANES):
                r = g * LANES + rr
                maskf = maskv[rr]
                for c in range(COL_CHUNKS):
                    sl = pl.ds(c * LANES, LANES)
                    rows_v[r, sl] = rows_v[r, sl] * maskf + pos_v[r, sl]
            return 0

        lax.fori_loop(0, ROWS_PER_W // LANES, group_body, 0)

        pltpu.sync_copy(rows_v, out_hbm.at[pl.ds(base, ROWS_PER_W)])

    return emb_kernel


_sc_kernel = _make_sc_kernel()


@jax.jit
def kernel(x0, x1, token_table, pos_table):
    x0_flat = x0.reshape(ROWS)
    out = _sc_kernel(x0_flat, token_table, pos_table)
    return out.reshape(BATCH, SEQ, EMBED_DIM), x1


# speedup vs baseline: 1.6865x; 1.6865x over previous
import functools

import jax
import jax.numpy as jnp
from jax import lax
from jax.experimental import pallas as pl
from jax.experimental.pallas import tpu as pltpu
from jax.experimental.pallas import tpu_sc as plsc

VOCAB = 1000000
EMBED_DIM = 64
MAX_SEQ = 2048
BATCH = 4
SEQ = 2048

NUM_CORES = 2
NUM_SUBCORES = 16
NUM_WORKERS = NUM_CORES * NUM_SUBCORES
ROWS = BATCH * SEQ
ROWS_PER_W = ROWS // NUM_WORKERS
T_PER_W = SEQ // (NUM_WORKERS // BATCH)
LANES = 16
COL_CHUNKS = EMBED_DIM // LANES
GROUPS = ROWS_PER_W // LANES


def _make_sc_kernel():
    mesh = plsc.VectorSubcoreMesh(core_axis_name="c", subcore_axis_name="s")

    @functools.partial(
        pl.kernel,
        mesh=mesh,
        out_type=jax.ShapeDtypeStruct((ROWS, EMBED_DIM), jnp.float32),
        scratch_types=[
            pltpu.VMEM((ROWS_PER_W,), jnp.int32),
            pltpu.VMEM((ROWS_PER_W, EMBED_DIM), jnp.float32),
            pltpu.VMEM((ROWS_PER_W, EMBED_DIM), jnp.float32),
            pltpu.SemaphoreType.DMA,
        ],
    )
    def emb_kernel(x0_hbm, table_hbm, pos_hbm, out_hbm, idx_v, rows_v, pos_v, sem):
        wid = lax.axis_index("s") * NUM_CORES + lax.axis_index("c")
        base = wid * ROWS_PER_W
        t0 = (wid % (NUM_WORKERS // BATCH)) * T_PER_W

        pltpu.sync_copy(x0_hbm.at[pl.ds(base, ROWS_PER_W)], idx_v)

        def group_fire(g, _):
            idxg = idx_v[pl.ds(g * LANES, LANES)]
            for j in range(LANES):
                row = idxg[j]
                pltpu.async_copy(
                    table_hbm.at[pl.ds(row, 1)],
                    rows_v.at[pl.ds(g * LANES + j, 1)],
                    sem,
                )
            return 0

        lax.fori_loop(0, GROUPS, group_fire, 0)

        pltpu.sync_copy(pos_hbm.at[pl.ds(t0, T_PER_W)], pos_v)

        @pl.when(t0 == 0)
        def _zero_pos_row0():
            for c in range(COL_CHUNKS):
                pos_v[0, pl.ds(c * LANES, LANES)] = jnp.zeros(
                    (LANES,), jnp.float32
                )

        drain = pltpu.make_async_copy(
            table_hbm.at[pl.ds(0, 1)], rows_v.at[pl.ds(0, 1)], sem
        )

        def group_drain(g, _):
            for _j in range(LANES):
                drain.wait()
            return 0

        lax.fori_loop(0, GROUPS, group_drain, 0)

        def group_body(g, _):
            idxg = idx_v[pl.ds(g * LANES, LANES)]
            maskv = jnp.where(idxg == 0, 0.0, 1.0).astype(jnp.float32)
            for rr in range(LANES):
                r = g * LANES + rr
                maskf = maskv[rr]
                for c in range(COL_CHUNKS):
                    sl = pl.ds(c * LANES, LANES)
                    rows_v[r, sl] = rows_v[r, sl] * maskf + pos_v[r, sl]
            return 0

        lax.fori_loop(0, GROUPS, group_body, 0)

        pltpu.sync_copy(rows_v, out_hbm.at[pl.ds(base, ROWS_PER_W)])

    return emb_kernel


_sc_kernel = _make_sc_kernel()


@jax.jit
def kernel(x0, x1, token_table, pos_table):
    x0_flat = x0.reshape(ROWS)
    out = _sc_kernel(x0_flat, token_table, pos_table)
    return out.reshape(BATCH, SEQ, EMBED_DIM), x1


# speedup vs baseline: 4.4878x; 2.6611x over previous
import functools

import jax
import jax.numpy as jnp
from jax import lax
from jax.experimental import pallas as pl
from jax.experimental.pallas import tpu as pltpu
from jax.experimental.pallas import tpu_sc as plsc

VOCAB = 1000000
EMBED_DIM = 64
MAX_SEQ = 2048
BATCH = 4
SEQ = 2048

NUM_CORES = 2
NUM_SUBCORES = 16
NUM_WORKERS = NUM_CORES * NUM_SUBCORES
ROWS = BATCH * SEQ
ROWS_PER_W = ROWS // NUM_WORKERS
T_PER_W = SEQ // (NUM_WORKERS // BATCH)
LANES = 16
COL_CHUNKS = EMBED_DIM // LANES
GROUPS = ROWS_PER_W // LANES
NSLOTS = 8


def _make_sc_kernel():
    mesh = plsc.VectorSubcoreMesh(core_axis_name="c", subcore_axis_name="s")

    @functools.partial(
        pl.kernel,
        mesh=mesh,
        compiler_params=pltpu.CompilerParams(needs_layout_passes=False),
        out_type=jax.ShapeDtypeStruct((ROWS, EMBED_DIM), jnp.float32),
        scratch_types=[
            pltpu.VMEM((ROWS_PER_W + LANES,), jnp.int32),
            pltpu.VMEM((NSLOTS * EMBED_DIM, 128), jnp.float32),
            pltpu.VMEM((2 * LANES, EMBED_DIM), jnp.float32),
            pltpu.VMEM((ROWS_PER_W, EMBED_DIM), jnp.float32),
            pltpu.SemaphoreType.DMA((NSLOTS,)),
            pltpu.SemaphoreType.DMA,
        ],
    )
    def emb_kernel(x0_hbm, tt_hbm, pos_hbm, out_hbm,
                   idx_v, slab_v, stage_v, pos_v, slab_sems, out_sem):
        wid = lax.axis_index("s") * NUM_CORES + lax.axis_index("c")
        base = wid * ROWS_PER_W
        t0 = (wid % (NUM_WORKERS // BATCH)) * T_PER_W
        iota = lax.iota(jnp.int32, LANES)
        zeros16i = jnp.zeros((LANES,), jnp.int32)

        pltpu.sync_copy(x0_hbm.at[pl.ds(base, ROWS_PER_W)],
                        idx_v.at[pl.ds(0, ROWS_PER_W)])
        idx_v[pl.ds(ROWS_PER_W, LANES)] = zeros16i

        pltpu.sync_copy(pos_hbm.at[pl.ds(t0, T_PER_W)], pos_v)

        @pl.when(t0 == 0)
        def _zero_pos_row0():
            for c in range(COL_CHUNKS):
                pos_v[0, pl.ds(c * LANES, LANES)] = jnp.zeros(
                    (LANES,), jnp.float32
                )

        def fire_slab(tok, slot):
            cstart = pl.multiple_of((tok // 128) * 128, 128)
            pltpu.async_copy(
                tt_hbm.at[:, pl.ds(cstart, 128)],
                slab_v.at[pl.ds(slot * EMBED_DIM, EMBED_DIM)],
                slab_sems.at[slot],
            )

        def wait_slab(slot):
            pltpu.make_async_copy(
                tt_hbm.at[:, pl.ds(0, 128)],
                slab_v.at[pl.ds(slot * EMBED_DIM, EMBED_DIM)],
                slab_sems.at[slot],
            ).wait()

        idxg0 = idx_v[pl.ds(0, LANES)]
        for j in range(NSLOTS):
            fire_slab(idxg0[j], j)

        out_drain = pltpu.make_async_copy(
            stage_v.at[pl.ds(0, LANES)],
            out_hbm.at[pl.ds(base, LANES)],
            out_sem,
        )

        def group_body(g, _):
            idxg = idx_v[pl.ds(g * LANES, LANES)]
            idxh = idx_v[pl.ds(g * LANES + LANES, LANES)]
            maskv = jnp.where(idxg == 0, 0.0, 1.0).astype(jnp.float32)
            p = (g % 2) * LANES

            @pl.when(g >= 2)
            def _wait_out():
                out_drain.wait()

            for j in range(LANES):
                slot = j % NSLOTS
                wait_slab(slot)
                i = idxg[j]
                lvec = (i % 128) + zeros16i
                maskf = maskv[j]
                for kc in range(COL_CHUNKS):
                    rowvec = slot * EMBED_DIM + kc * LANES + iota
                    vals = plsc.load_gather(slab_v, [rowvec, lvec])
                    stage_v[p + j, pl.ds(kc * LANES, LANES)] = (
                        vals * maskf + pos_v[g * LANES + j, pl.ds(kc * LANES, LANES)]
                    )
                nid = idxg[j + NSLOTS] if j < NSLOTS else idxh[j - NSLOTS]
                fire_slab(nid, slot)

            pltpu.async_copy(
                stage_v.at[pl.ds(p, LANES)],
                out_hbm.at[pl.ds(base + g * LANES, LANES)],
                out_sem,
            )
            return 0

        lax.fori_loop(0, GROUPS, group_body, 0)

        for s in range(NSLOTS):
            wait_slab(s)
        out_drain.wait()
        out_drain.wait()

    return emb_kernel


_sc_kernel = _make_sc_kernel()


@jax.jit
def kernel(x0, x1, token_table, pos_table):
    x0_flat = x0.reshape(ROWS)
    out = _sc_kernel(x0_flat, token_table.T, pos_table)
    return out.reshape(BATCH, SEQ, EMBED_DIM), x1
